# baseline (speedup 1.0000x reference)
import functools

import jax
import jax.numpy as jnp
from jax import lax
from jax.experimental import pallas as pl
from jax.experimental.pallas import tpu as pltpu
from jax.experimental.pallas import tpu_sc as plsc

_B, _F, _V = 4, 128, 12288
_ROWS = _B * _F
_V4 = 4 * _V
_NW = 32
_LANES = 16
_UNROLL = 4

_R_SC = 128
_R_TC = _ROWS - _R_SC
_RPW = _R_SC // _NW

_mesh = plsc.VectorSubcoreMesh(core_axis_name="c", subcore_axis_name="s")


@functools.partial(
    pl.kernel,
    out_type=jax.ShapeDtypeStruct((_R_SC, _V4), jnp.float32),
    mesh=_mesh,
    compiler_params=pltpu.CompilerParams(needs_layout_passes=False),
    scratch_types=[
        pltpu.VMEM((_V,), jnp.float32),
        pltpu.VMEM((_V,), jnp.float32),
        pltpu.VMEM((_V4,), jnp.float32),
        pltpu.VMEM((_V4,), jnp.float32),
        pltpu.SemaphoreType.DMA,
        pltpu.SemaphoreType.DMA,
        pltpu.SemaphoreType.DMA,
        pltpu.SemaphoreType.DMA,
    ],
)
def _unpool_sc(x_hbm, out_hbm, in0, in1, out0, out1, si0, si1, so0, so1):
    wid = lax.axis_index("s") * 2 + lax.axis_index("c")
    row0 = wid * _RPW
    ins = (in0, in1)
    outs = (out0, out1)
    sins = (si0, si1)
    souts = (so0, so1)

    in_h = [None, None]
    out_h = [None, None]
    in_h[0] = pltpu.async_copy(x_hbm.at[row0 + 0], in0, si0)
    if _RPW > 1:
        in_h[1] = pltpu.async_copy(x_hbm.at[row0 + 1], in1, si1)
    for r in range(_RPW):
        b = r % 2
        in_h[b].wait()
        if r >= 2:
            out_h[b].wait()
        src = ins[b]
        dst = outs[b]

        @plsc.parallel_loop(0, _V4 // _LANES, step=4, unroll=_UNROLL)
        def body(j, src=src, dst=dst):
            iota4 = lax.iota(jnp.int32, _LANES) // 4
            for q in range(4):
                vals = plsc.load_gather(src, [iota4 + (4 * (j + q))])
                dst[pl.ds((j + q) * _LANES, _LANES)] = vals

        out_h[b] = pltpu.async_copy(dst, out_hbm.at[row0 + r], souts[b])
        if r + 2 < _RPW:
            in_h[b] = pltpu.async_copy(x_hbm.at[row0 + r + 2], ins[b], sins[b])
    out_h[0].wait()
    if _RPW > 1:
        out_h[1].wait()


_R_BLK = 128
_C_BLK = 128


def _expand_tc_body(x_ref, e_ref, o_ref):
    o_ref[...] = jnp.dot(x_ref[...], e_ref[...],
                         preferred_element_type=jnp.float32)


def _merge_body(sc_ref, tc_ref, o_ref):
    o_ref[...] = sc_ref[...]


def _expand_tc(x2, e):
    return pl.pallas_call(
        _expand_tc_body,
        grid=(_R_TC // _R_BLK, _V // _C_BLK),
        in_specs=[
            pl.BlockSpec((_R_BLK, _C_BLK),
                         lambda i, j: (i + _R_SC // _R_BLK, j)),
            pl.BlockSpec((_C_BLK, 4 * _C_BLK), lambda i, j: (0, 0)),
        ],
        out_specs=pl.BlockSpec((_R_BLK, 4 * _C_BLK),
                               lambda i, j: (i + _R_SC // _R_BLK, j)),
        out_shape=jax.ShapeDtypeStruct((_ROWS, _V4), jnp.float32),
    )(x2, e)


_MR_BLK = 64
_MC_BLK = 12288


def _merge(sc_out, tc_out):
    return pl.pallas_call(
        _merge_body,
        grid=(_R_SC // _MR_BLK, _V4 // _MC_BLK),
        in_specs=[
            pl.BlockSpec((_MR_BLK, _MC_BLK), lambda i, j: (i, j)),
            pl.BlockSpec(memory_space=pl.ANY),
        ],
        out_specs=pl.BlockSpec((_MR_BLK, _MC_BLK), lambda i, j: (i, j)),
        out_shape=jax.ShapeDtypeStruct((_ROWS, _V4), jnp.float32),
        input_output_aliases={1: 0},
    )(sc_out, tc_out)


def _make_e():
    i = lax.broadcasted_iota(jnp.int32, (_C_BLK, 4 * _C_BLK), 0)
    j = lax.broadcasted_iota(jnp.int32, (_C_BLK, 4 * _C_BLK), 1)
    return (j // 4 == i).astype(jnp.float32)


def kernel(x, indices_spa, indices_sph):
    x2 = x.reshape(_ROWS, _V)
    sc_out = _unpool_sc(x2[:_R_SC])
    tc_out = _expand_tc(x2, _make_e())
    out = _merge(sc_out, tc_out)
    return out.reshape(_B, _F, _V4, 1, 1, 1)


# speedup vs baseline: 1.3062x; 1.3062x over previous
import functools

import jax
import jax.numpy as jnp
from jax import lax
from jax.experimental import pallas as pl
from jax.experimental.pallas import tpu as pltpu
from jax.experimental.pallas import tpu_sc as plsc

_B, _F, _V = 4, 128, 12288
_ROWS = _B * _F
_V4 = 4 * _V
_NW = 32
_LANES = 16
_UNROLL = 4

_R_SC = 128
_R_TC = _ROWS - _R_SC
_RPW = _R_SC // _NW

_mesh = plsc.VectorSubcoreMesh(core_axis_name="c", subcore_axis_name="s")


@functools.partial(
    pl.kernel,
    out_type=jax.ShapeDtypeStruct((_R_SC, _V4), jnp.float32),
    mesh=_mesh,
    compiler_params=pltpu.CompilerParams(needs_layout_passes=False),
    scratch_types=[
        pltpu.VMEM((_V,), jnp.float32),
        pltpu.VMEM((_V,), jnp.float32),
        pltpu.VMEM((_V4,), jnp.float32),
        pltpu.VMEM((_V4,), jnp.float32),
        pltpu.SemaphoreType.DMA,
        pltpu.SemaphoreType.DMA,
        pltpu.SemaphoreType.DMA,
        pltpu.SemaphoreType.DMA,
    ],
)
def _unpool_sc(x_hbm, out_hbm, in0, in1, out0, out1, si0, si1, so0, so1):
    wid = lax.axis_index("s") * 2 + lax.axis_index("c")
    row0 = wid * _RPW
    ins = (in0, in1)
    outs = (out0, out1)
    sins = (si0, si1)
    souts = (so0, so1)

    in_h = [None, None]
    out_h = [None, None]
    in_h[0] = pltpu.async_copy(x_hbm.at[row0 + 0], in0, si0)
    if _RPW > 1:
        in_h[1] = pltpu.async_copy(x_hbm.at[row0 + 1], in1, si1)
    for r in range(_RPW):
        b = r % 2
        in_h[b].wait()
        if r >= 2:
            out_h[b].wait()
        src = ins[b]
        dst = outs[b]

        @plsc.parallel_loop(0, _V4 // _LANES, step=4, unroll=_UNROLL)
        def body(j, src=src, dst=dst):
            iota4 = lax.iota(jnp.int32, _LANES) // 4
            for q in range(4):
                vals = plsc.load_gather(src, [iota4 + (4 * (j + q))])
                dst[pl.ds((j + q) * _LANES, _LANES)] = vals

        out_h[b] = pltpu.async_copy(dst, out_hbm.at[row0 + r], souts[b])
        if r + 2 < _RPW:
            in_h[b] = pltpu.async_copy(x_hbm.at[row0 + r + 2], ins[b], sins[b])
    out_h[0].wait()
    if _RPW > 1:
        out_h[1].wait()


_R_BLK = 128
_C_BLK = 512


def _expand_tc_body(sc_ref, x_ref, e_ref, o_ref, sem):
    i = pl.program_id(0)

    @pl.when(i == 0)
    def _():
        j = pl.program_id(1)
        pltpu.make_async_copy(
            sc_ref.at[:, pl.ds(j * 4 * _C_BLK, 4 * _C_BLK)], o_ref, sem,
        ).start()
        pltpu.make_async_copy(
            sc_ref.at[:, pl.ds(j * 4 * _C_BLK, 4 * _C_BLK)], o_ref, sem,
        ).wait()

    @pl.when(i > 0)
    def _():
        for c in range(_C_BLK // 128):
            o_ref[:, c * 512:(c + 1) * 512] = jnp.dot(
                x_ref[:, c * 128:(c + 1) * 128], e_ref[...],
                preferred_element_type=jnp.float32)


def _expand_tc(sc_out, x2, e):
    return pl.pallas_call(
        _expand_tc_body,
        grid=(_ROWS // _R_BLK, _V // _C_BLK),
        in_specs=[
            pl.BlockSpec(memory_space=pl.ANY),
            pl.BlockSpec((_R_BLK, _C_BLK), lambda i, j: (i, j)),
            pl.BlockSpec((128, 512), lambda i, j: (0, 0)),
        ],
        out_specs=pl.BlockSpec((_R_BLK, 4 * _C_BLK), lambda i, j: (i, j)),
        out_shape=jax.ShapeDtypeStruct((_ROWS, _V4), jnp.float32),
        scratch_shapes=[pltpu.SemaphoreType.DMA],
    )(sc_out, x2, e)


def _make_e():
    i = lax.broadcasted_iota(jnp.int32, (128, 512), 0)
    j = lax.broadcasted_iota(jnp.int32, (128, 512), 1)
    return (j // 4 == i).astype(jnp.float32)


def kernel(x, indices_spa, indices_sph):
    x2 = x.reshape(_ROWS, _V)
    sc_out = _unpool_sc(x2[:_R_SC])
    out = _expand_tc(sc_out, x2, _make_e())
    return out.reshape(_B, _F, _V4, 1, 1, 1)


# speedup vs baseline: 1.5035x; 1.1510x over previous
import functools

import jax
import jax.numpy as jnp
from jax import lax
from jax.experimental import pallas as pl
from jax.experimental.pallas import tpu as pltpu
from jax.experimental.pallas import tpu_sc as plsc

_B, _F, _V = 4, 128, 12288
_ROWS = _B * _F
_V4 = 4 * _V
_NW = 32
_LANES = 16
_UNROLL = 4

_R_SC = 128
_R_TC = _ROWS - _R_SC
_RPW = _R_SC // _NW

_mesh = plsc.VectorSubcoreMesh(core_axis_name="c", subcore_axis_name="s")


@functools.partial(
    pl.kernel,
    out_type=jax.ShapeDtypeStruct((_R_SC, _V4), jnp.float32),
    mesh=_mesh,
    compiler_params=pltpu.CompilerParams(needs_layout_passes=False),
    scratch_types=[
        pltpu.VMEM((_V,), jnp.float32),
        pltpu.VMEM((_V,), jnp.float32),
        pltpu.VMEM((_V4,), jnp.float32),
        pltpu.VMEM((_V4,), jnp.float32),
        pltpu.SemaphoreType.DMA,
        pltpu.SemaphoreType.DMA,
        pltpu.SemaphoreType.DMA,
        pltpu.SemaphoreType.DMA,
    ],
)
def _unpool_sc(x_hbm, out_hbm, in0, in1, out0, out1, si0, si1, so0, so1):
    wid = lax.axis_index("s") * 2 + lax.axis_index("c")
    row0 = wid * _RPW
    ins = (in0, in1)
    outs = (out0, out1)
    sins = (si0, si1)
    souts = (so0, so1)

    in_h = [None, None]
    out_h = [None, None]
    in_h[0] = pltpu.async_copy(x_hbm.at[row0 + 0], in0, si0)
    if _RPW > 1:
        in_h[1] = pltpu.async_copy(x_hbm.at[row0 + 1], in1, si1)
    for r in range(_RPW):
        b = r % 2
        in_h[b].wait()
        if r >= 2:
            out_h[b].wait()
        src = ins[b]
        dst = outs[b]

        @plsc.parallel_loop(0, _V4 // _LANES, step=4, unroll=_UNROLL)
        def body(j, src=src, dst=dst):
            iota4 = lax.iota(jnp.int32, _LANES) // 4
            for q in range(4):
                vals = plsc.load_gather(src, [iota4 + (4 * (j + q))])
                dst[pl.ds((j + q) * _LANES, _LANES)] = vals

        out_h[b] = pltpu.async_copy(dst, out_hbm.at[row0 + r], souts[b])
        if r + 2 < _RPW:
            in_h[b] = pltpu.async_copy(x_hbm.at[row0 + r + 2], ins[b], sins[b])
    out_h[0].wait()
    if _RPW > 1:
        out_h[1].wait()


_C_BLK = 1024


def _expand_tc_body(sc_ref, x_ref, e_ref, o_ref, sem):
    i = pl.program_id(0)

    @pl.when(i == 0)
    def _():
        j = pl.program_id(1)
        pltpu.make_async_copy(
            sc_ref.at[:, pl.ds(j * 4 * _C_BLK, 4 * _C_BLK)], o_ref.at[0], sem,
        ).start()
        pltpu.make_async_copy(
            sc_ref.at[:, pl.ds(j * 4 * _C_BLK, 4 * _C_BLK)], o_ref.at[0], sem,
        ).wait()

    @pl.when(i > 0)
    def _():
        for c in range(_C_BLK // 128):
            o_ref[0, :, c * 512:(c + 1) * 512] = jnp.dot(
                x_ref[0, :, c * 128:(c + 1) * 128], e_ref[...],
                preferred_element_type=jnp.float32)


def _expand_tc(sc_out, x3, e):
    return pl.pallas_call(
        _expand_tc_body,
        grid=(_B, _V // _C_BLK),
        in_specs=[
            pl.BlockSpec(memory_space=pl.ANY),
            pl.BlockSpec((1, _F, _C_BLK), lambda i, j: (i, 0, j)),
            pl.BlockSpec((128, 512), lambda i, j: (0, 0)),
        ],
        out_specs=pl.BlockSpec((1, _F, 4 * _C_BLK), lambda i, j: (i, 0, j)),
        out_shape=jax.ShapeDtypeStruct((_B, _F, _V4), jnp.float32),
        scratch_shapes=[pltpu.SemaphoreType.DMA],
    )(sc_out, x3, e)


def _make_e():
    i = lax.broadcasted_iota(jnp.int32, (128, 512), 0)
    j = lax.broadcasted_iota(jnp.int32, (128, 512), 1)
    return (j // 4 == i).astype(jnp.float32)


def kernel(x, indices_spa, indices_sph):
    x3 = lax.squeeze(x, (3, 4, 5))
    x2 = x3.reshape(_ROWS, _V)
    sc_out = _unpool_sc(x2[:_R_SC])
    out = _expand_tc(sc_out, x3, _make_e())
    return lax.expand_dims(out, (3, 4, 5))


# speedup vs baseline: 4.4376x; 2.9515x over previous
import functools

import jax
import jax.numpy as jnp
from jax import lax
from jax.experimental import pallas as pl
from jax.experimental.pallas import tpu as pltpu
from jax.experimental.pallas import tpu_sc as plsc

_B, _F, _V = 4, 128, 12288
_ROWS = _B * _F
_V4 = 4 * _V
_NW = 32
_RPW = _ROWS // _NW
_LANES = 16
_UNROLL = 4

_mesh = plsc.VectorSubcoreMesh(core_axis_name="c", subcore_axis_name="s")


@functools.partial(
    pl.kernel,
    out_type=jax.ShapeDtypeStruct((_ROWS, _V4), jnp.float32),
    mesh=_mesh,
    compiler_params=pltpu.CompilerParams(needs_layout_passes=False, use_tc_tiling_on_sc=False),
    scratch_types=[
        pltpu.VMEM((_V,), jnp.float32),
        pltpu.VMEM((_V,), jnp.float32),
        pltpu.VMEM((_V4,), jnp.float32),
        pltpu.VMEM((_V4,), jnp.float32),
        pltpu.SemaphoreType.DMA,
        pltpu.SemaphoreType.DMA,
        pltpu.SemaphoreType.DMA,
        pltpu.SemaphoreType.DMA,
    ],
)
def _unpool_sc(x_hbm, out_hbm, in0, in1, out0, out1, si0, si1, so0, so1):
    wid = lax.axis_index("s") * 2 + lax.axis_index("c")
    row0 = wid * _RPW
    ins = (in0, in1)
    outs = (out0, out1)
    sins = (si0, si1)
    souts = (so0, so1)

    in_h = [None, None]
    out_h = [None, None]
    in_h[0] = pltpu.async_copy(x_hbm.at[row0 + 0], in0, si0)
    in_h[1] = pltpu.async_copy(x_hbm.at[row0 + 1], in1, si1)
    for r in range(_RPW):
        b = r % 2
        in_h[b].wait()
        if r >= 2:
            out_h[b].wait()
        src = ins[b]
        dst = outs[b]

        @plsc.parallel_loop(0, _V4 // _LANES, step=4, unroll=_UNROLL)
        def body(j, src=src, dst=dst):
            iota4 = lax.iota(jnp.int32, _LANES) // 4
            for q in range(4):
                vals = plsc.load_gather(src, [iota4 + (4 * (j + q))])
                dst[pl.ds((j + q) * _LANES, _LANES)] = vals
        out_h[b] = pltpu.async_copy(dst, out_hbm.at[row0 + r], souts[b])
        if r + 2 < _RPW:
            in_h[b] = pltpu.async_copy(x_hbm.at[row0 + r + 2], ins[b], sins[b])
    out_h[0].wait()
    out_h[1].wait()


def kernel(x, indices_spa, indices_sph):
    x2 = x.reshape(_ROWS, _V)
    out = _unpool_sc(x2)
    return out.reshape(_B, _F, _V4, 1, 1, 1)
